# Initial kernel scaffold; baseline (speedup 1.0000x reference)
import jax
import jax.numpy as jnp
from jax.experimental import pallas as pl


def kernel(x1, x2):
    raise NotImplementedError("write your pallas kernel here")


# speedup vs baseline: 3.3936x; 3.3936x over previous
import functools

import jax
import jax.numpy as jnp
from jax import lax
from jax.experimental import pallas as pl
from jax.experimental.pallas import tpu as pltpu
from jax.experimental.pallas import tpu_sc as plsc

_N = 12544
_C = 64
_M = 2205
_K = 5
_BN = 896
_NB = _N // _BN
_NR = 98
_SEL = 250
_SELP = 256
_GB = 1280
_NW = 32
_BPW = _GB // _NW
_GC = 128


def _scores_body(xn_ref, sn_ref, out_ref):
    x = xn_ref[...]
    s = sn_ref[0]
    sim = jnp.dot(x, s, preferred_element_type=jnp.float32)
    v1 = jnp.max(sim, axis=1, keepdims=True)
    e1 = sim == v1
    n1 = jnp.sum(e1.astype(jnp.int32), axis=1, keepdims=True)
    sim = jnp.where(e1, -jnp.inf, sim)
    v2 = jnp.max(sim, axis=1, keepdims=True)
    e2 = sim == v2
    n2 = jnp.sum(e2.astype(jnp.int32), axis=1, keepdims=True)
    sim = jnp.where(e2, -jnp.inf, sim)
    v3 = jnp.max(sim, axis=1, keepdims=True)
    b = jnp.where(n1 >= 2, v1, v2)
    c = jnp.where(n1 >= 3, v1, jnp.where(n1 + n2 >= 3, v2, v3))
    res = (v1 + c) + b
    j = pl.program_id(1)
    col = lax.broadcasted_iota(jnp.int32, (_BN, _K), 1)

    @pl.when(j == 0)
    def _init():
        out_ref[...] = jnp.broadcast_to(res, (_BN, _K))

    @pl.when(j > 0)
    def _update():
        out_ref[...] = jnp.where(col == j, res, out_ref[...])


def _select_body(s_ref, idx_ref):
    s0 = s_ref[0]
    rid = lax.broadcasted_iota(jnp.int32, (_NR, 128), 0)
    cid = lax.broadcasted_iota(jnp.int32, (_NR, 128), 1)
    nid = rid * 128 + cid
    arow = lax.broadcasted_iota(jnp.int32, (2, 128), 0)
    acol = lax.broadcasted_iota(jnp.int32, (2, 128), 1)

    def body(t, carry):
        s, acc = carry
        vmax = jnp.max(s)
        cand = jnp.where(s == vmax, nid, _N)
        imin = jnp.min(cand)
        s = jnp.where(nid == imin, -jnp.inf, s)
        acc = jnp.where((arow == t // 128) & (acol == t % 128), imin, acc)
        return (s, acc)

    _, acc = lax.fori_loop(0, _SEL, body, (s0, jnp.zeros((2, 128), jnp.int32)))
    idx_ref[0] = acc


@functools.cache
def _make_sc_gather():
    mesh = plsc.VectorSubcoreMesh(core_axis_name="c", subcore_axis_name="s")

    @functools.partial(
        pl.kernel,
        mesh=mesh,
        out_type=jax.ShapeDtypeStruct((_GB, _GC), jnp.float32),
        scratch_types=[
            pltpu.VMEM((_BPW,), jnp.int32),
            pltpu.VMEM((_BPW, _GC), jnp.float32),
            pltpu.SemaphoreType.DMA,
        ],
    )
    def _sc_gather(table_hbm, idx_hbm, out_hbm, idx_v, rows_v, sem):
        wid = lax.axis_index("s") * 2 + lax.axis_index("c")
        base = wid * _BPW
        pltpu.sync_copy(idx_hbm.at[pl.ds(base, _BPW)], idx_v)
        pltpu.async_copy(table_hbm.at[idx_v], rows_v, sem).wait()
        pltpu.sync_copy(rows_v, out_hbm.at[pl.ds(base, _BPW)])

    return _sc_gather


def kernel(x1, x2):
    B, C, h, w = x1.shape
    K = x2.shape[0]
    raw_t = jnp.transpose(x1, (1, 0, 2, 3)).reshape(C, -1).T
    norm = jnp.linalg.norm(raw_t, ord=2, axis=1, keepdims=True)
    xn = raw_t / norm
    sn = jnp.stack(
        [x2[j] / jnp.linalg.norm(x2[j], ord=2, axis=0, keepdims=True)
         for j in range(K)]
    )

    scores_t = pl.pallas_call(
        _scores_body,
        grid=(_NB, _K),
        in_specs=[
            pl.BlockSpec((_BN, _C), lambda nb, j: (nb, 0)),
            pl.BlockSpec((1, _C, _M), lambda nb, j: (j, 0, 0)),
        ],
        out_specs=pl.BlockSpec((_BN, _K), lambda nb, j: (nb, 0)),
        out_shape=jax.ShapeDtypeStruct((_N, _K), jnp.float32),
    )(xn, sn)

    scores = scores_t.T.reshape(_K, _NR, 128)
    idx = pl.pallas_call(
        _select_body,
        grid=(_K,),
        in_specs=[pl.BlockSpec((1, _NR, 128), lambda j: (j, 0, 0))],
        out_specs=pl.BlockSpec((1, 2, 128), lambda j: (j, 0, 0)),
        out_shape=jax.ShapeDtypeStruct((_K, 2, 128), jnp.int32),
    )(scores)

    flat_idx = idx.reshape(_K, _SELP)[:, :_SEL].reshape(-1)
    flat_idx = jnp.concatenate(
        [flat_idx, jnp.zeros((_GB - _K * _SEL,), jnp.int32)]
    )
    table = jnp.pad(raw_t, ((0, 0), (0, _GC - _C)))
    rows = _make_sc_gather()(table, flat_idx)
    sel = rows[: _K * _SEL, :_C].reshape(_K, _SEL, C).transpose(0, 2, 1)
    return jnp.concatenate([x2, sel], axis=2)
